# baseline (speedup 1.0000x reference)
import functools

import jax
import jax.numpy as jnp
from jax import lax
from jax.experimental import pallas as pl
from jax.experimental.pallas import tpu as pltpu
from jax.experimental.pallas import tpu_sc as plsc

NC = 2
NS = 16
L = 16

BN = 400
CH = 128
SUP = 6
NBUF = 3


def _dense_layer1_body(texts, nf, Wt, bt, Wg01, bg01, Wg02, bg02,
                       Wg11, bg11, Wg12, bg12, W0, b0,
                       cur_out, s2_out):
    te = jnp.dot(texts[...], Wt[...], preferred_element_type=jnp.float32) + bt[...]
    t1 = jax.nn.relu(jnp.dot(te, Wg01[...], preferred_element_type=jnp.float32) + bg01[...])
    s1 = jax.nn.sigmoid(jnp.dot(t1, Wg02[...], preferred_element_type=jnp.float32) + bg02[...])
    c = (jnp.dot(nf[...], W0[...], preferred_element_type=jnp.float32) + b0[...]) * s1
    cur_out[...] = c.reshape(cur_out.shape)
    t2 = jax.nn.relu(jnp.dot(te, Wg11[...], preferred_element_type=jnp.float32) + bg11[...])
    s2_out[...] = jax.nn.sigmoid(jnp.dot(t2, Wg12[...], preferred_element_type=jnp.float32) + bg12[...])


def _dense_layer2_body(cur, agg_a, agg_b, deg, s2, W1, b1, cur2_out):
    bn = deg.shape[0]
    rdeg = 1.0 / jnp.maximum(deg[...], 1.0)
    agg = jnp.concatenate([agg_a[...], agg_b[...]], axis=1)
    c = cur[...].reshape(bn, -1)
    h = jax.nn.relu((c + agg * rdeg) * 0.5)
    c2 = (jnp.dot(h, W1[...], preferred_element_type=jnp.float32) + b1[...]) * s2[...]
    cur2_out[...] = c2.reshape(cur2_out.shape)


def _combine_body(cur, agg_a, agg_b, deg, out):
    bn = deg.shape[0]
    rdeg = 1.0 / jnp.maximum(deg[...], 1.0)
    agg = jnp.concatenate([agg_a[...], agg_b[...]], axis=1)
    out[...] = (cur[...].reshape(bn, -1) + agg * rdeg) * 0.5


def _row_spec(bshape):
    nd = len(bshape)
    return pl.BlockSpec(bshape, lambda i, _nd=nd: (i,) + (0,) * (_nd - 1))


def _full_spec(shape):
    nd = len(shape)
    return pl.BlockSpec(shape, lambda i, _nd=nd: (0,) * _nd)


_SC_PARAMS = pltpu.CompilerParams(use_tc_tiling_on_sc=False,
                                  needs_layout_passes=False)


def _mesh():
    return plsc.VectorSubcoreMesh(core_axis_name="c", subcore_axis_name="s",
                                  num_cores=NC, num_subcores=NS)


def _make_sc_degree(n, e, hh):
    ept = e // NS
    assert e % (NS * L) == 0
    degp = ((n + NS * hh - 1) // (NS * hh)) * NS * hh
    dgr = degp // hh
    drt = dgr // NS
    half = NS // 2

    @functools.partial(
        pl.kernel,
        out_type=jax.ShapeDtypeStruct((dgr, hh), jnp.float32),
        mesh=_mesh(),
        scratch_types=[
            pltpu.VMEM((ept,), jnp.int32),
            pltpu.VMEM((dgr, hh), jnp.float32),
            pltpu.VMEM((half * drt, hh), jnp.float32),
            pltpu.VMEM((drt, hh), jnp.float32),
            pltpu.HBM((NS * dgr, hh), jnp.float32),
        ],
        compiler_params=_SC_PARAMS,
    )
    def sc_degree(col_hbm, deg_hbm, colbuf, deg_v, red0, red1, deg_sh):
        cid = lax.axis_index("c")
        sid = lax.axis_index("s")
        zeros16 = jnp.zeros((L,), jnp.float32)
        ones16 = jnp.ones((L,), jnp.float32)

        @pl.when(cid == 0)
        def _():
            def _zdeg(i, _):
                for j in range(hh // L):
                    deg_v[i, pl.ds(j * L, L)] = zeros16
                return 0
            lax.fori_loop(0, dgr, _zdeg, 0)
            pltpu.sync_copy(col_hbm.at[pl.ds(sid * ept, ept)], colbuf)

            def _hist(i, _):
                for u in range(5):
                    c = colbuf[pl.ds((i * 5 + u) * L, L)]
                    plsc.addupdate_scatter(
                        deg_v,
                        [lax.shift_right_logical(c, 7),
                         lax.bitwise_and(c, 127)],
                        ones16)
                return 0
            lax.fori_loop(0, ept // (5 * L), _hist, 0)
            pltpu.sync_copy(deg_v, deg_sh.at[pl.ds(sid * dgr, dgr)])
            plsc.subcore_barrier()

            for g in range(2):
                for t8 in range(half):
                    tt = g * half + t8
                    pltpu.sync_copy(
                        deg_sh.at[pl.ds(tt * dgr + sid * drt, drt)],
                        red0.at[pl.ds(t8 * drt, drt)])
                for p in range(drt):
                    for q in range(hh // L):
                        acc = red0[p, pl.ds(q * L, L)]
                        for t8 in range(1, half):
                            acc = acc + red0[t8 * drt + p, pl.ds(q * L, L)]
                        if g == 0:
                            red1[p, pl.ds(q * L, L)] = acc
                        else:
                            red1[p, pl.ds(q * L, L)] = (
                                red1[p, pl.ds(q * L, L)] + acc)
                pltpu.sync_copy(red1, deg_hbm.at[pl.ds(sid * drt, drt)])

    return sc_degree


def _make_sc_aggregate(n, epad, hh):
    cpt = epad // (NS * CH)
    nsup = cpt // SUP
    rem = cpt % SUP
    assert epad % (NS * CH) == 0 and n % NS == 0 and hh == 128
    rpt = n // NS

    @functools.partial(
        pl.kernel,
        out_type=(
            jax.ShapeDtypeStruct((n, hh), jnp.float32),
            jax.ShapeDtypeStruct((n, hh), jnp.float32),
        ),
        mesh=_mesh(),
        scratch_types=(
            [pltpu.VMEM((SUP, CH), jnp.int32),
             pltpu.VMEM((SUP, CH), jnp.int32)]
            + [pltpu.VMEM((CH, hh), jnp.float32) for _ in range(NBUF)]
            + [pltpu.VMEM_SHARED((n + 8, hh), jnp.float32)]
            + [pltpu.SemaphoreType.DMA for _ in range(2 * NBUF)]
        ),
        compiler_params=_SC_PARAMS,
    )
    def sc_aggregate(row_hbm, col_hbm, cur_hbm, dep_hbm, agg_a_hbm,
                     agg_b_hbm, *rest):
        del dep_hbm
        rowbuf, colbuf = rest[0], rest[1]
        rb = rest[2:2 + NBUF]
        agg_sh = rest[2 + NBUF]
        gsem = rest[3 + NBUF:3 + 2 * NBUF]
        ssem = rest[3 + 2 * NBUF:3 + 3 * NBUF]
        cid = lax.axis_index("c")
        sid = lax.axis_index("s")
        zeros16 = jnp.zeros((L,), jnp.float32)

        def _zrow(i, _):
            for j in range(hh // L):
                rb[0][i, pl.ds(j * L, L)] = zeros16
            return 0
        lax.fori_loop(0, CH, _zrow, 0)
        done = 0
        while done < rpt:
            step = min(CH, rpt - done)
            pltpu.sync_copy(rb[0].at[pl.ds(0, step)],
                            agg_sh.at[pl.ds(sid * rpt + done, step)])
            done += step
        plsc.subcore_barrier()

        def _run_super(base, count):
            pltpu.sync_copy(row_hbm.at[pl.ds(base, count)],
                            rowbuf.at[pl.ds(0, count)])
            pltpu.sync_copy(col_hbm.at[pl.ds(base, count)],
                            colbuf.at[pl.ds(0, count)])
            for j in range(count):
                for l in range(CH // L):
                    r = rowbuf[j, pl.ds(l * L, L)]
                    rowbuf[j, pl.ds(l * L, L)] = r + r + cid
            gd = [None] * NBUF
            sd = [None] * NBUF
            for j in range(min(NBUF - 1, count)):
                gd[j] = pltpu.async_copy(cur_hbm.at[rowbuf.at[j]], rb[j],
                                         gsem[j])
            for j in range(count):
                b = j % NBUF
                gd[b].wait()
                if j + NBUF - 1 < count:
                    nb = (j + NBUF - 1) % NBUF
                    if sd[nb] is not None:
                        sd[nb].wait()
                    gd[nb] = pltpu.async_copy(
                        cur_hbm.at[rowbuf.at[j + NBUF - 1]], rb[nb], gsem[nb])
                sd[b] = pltpu.async_copy(rb[b], agg_sh.at[colbuf.at[j]],
                                         ssem[b], add=True)
            for b in range(NBUF):
                if sd[b] is not None:
                    sd[b].wait()

        def _super(s, _):
            _run_super(sid * cpt + s * SUP, SUP)
            return 0
        lax.fori_loop(0, nsup, _super, 0)
        if rem:
            _run_super(sid * cpt + nsup * SUP, rem)
        plsc.subcore_barrier()

        @pl.when(cid == 0)
        def _():
            pltpu.sync_copy(agg_sh.at[pl.ds(sid * rpt, rpt)],
                            agg_a_hbm.at[pl.ds(sid * rpt, rpt)])

        @pl.when(cid == 1)
        def _():
            pltpu.sync_copy(agg_sh.at[pl.ds(sid * rpt, rpt)],
                            agg_b_hbm.at[pl.ds(sid * rpt, rpt)])

    return sc_aggregate


def kernel(edge_index, node_features, node_texts, Wt, bt, Wg01, bg01, Wg02,
           bg02, Wg11, bg11, Wg12, bg12, W0, b0, W1, b1):
    n, d = node_features.shape
    t = node_texts.shape[1]
    h = W0.shape[1]
    e = edge_index.shape[1]
    hh = h // NC
    assert n % BN == 0
    nblk = n // BN

    row = edge_index[0]
    col = edge_index[1]
    epad = ((e + NS * CH - 1) // (NS * CH)) * NS * CH
    if epad != e:
        rowp = jnp.concatenate([row, jnp.zeros((epad - e,), jnp.int32)])
        colp = jnp.concatenate([col, jnp.full((epad - e,), n, jnp.int32)])
    else:
        rowp, colp = row, col
    row2d = rowp.reshape(epad // CH, CH)
    col2d = colp.reshape(epad // CH, CH)
    b2 = lambda b: b.reshape(1, -1)

    sc_degree = _make_sc_degree(n, e, hh)
    deg = sc_degree(col)
    deg2d = deg.reshape(-1)[:n].reshape(n, 1)

    cur1, s2 = pl.pallas_call(
        _dense_layer1_body,
        grid=(nblk,),
        in_specs=[
            _row_spec((BN, t)), _row_spec((BN, d)),
            _full_spec((t, h)), _full_spec((1, h)),
            _full_spec((h, h)), _full_spec((1, h)),
            _full_spec((h, h)), _full_spec((1, h)),
            _full_spec((h, h)), _full_spec((1, h)),
            _full_spec((h, h)), _full_spec((1, h)),
            _full_spec((d, h)), _full_spec((1, h)),
        ],
        out_specs=[_row_spec((BN, 2, hh)), _row_spec((BN, h))],
        out_shape=[
            jax.ShapeDtypeStruct((n, 2, hh), jnp.float32),
            jax.ShapeDtypeStruct((n, h), jnp.float32),
        ],
    )(node_texts, node_features, Wt, b2(bt), Wg01, b2(bg01), Wg02, b2(bg02),
      Wg11, b2(bg11), Wg12, b2(bg12), W0, b2(b0))

    sc_aggregate = _make_sc_aggregate(n, epad, hh)

    agg1_a, agg1_b = sc_aggregate(row2d, col2d, cur1.reshape(2 * n, hh), deg)

    cur2 = pl.pallas_call(
        _dense_layer2_body,
        grid=(nblk,),
        in_specs=[
            _row_spec((BN, 2, hh)), _row_spec((BN, hh)),
            _row_spec((BN, hh)), _row_spec((BN, 1)), _row_spec((BN, h)),
            _full_spec((h, h)), _full_spec((1, h)),
        ],
        out_specs=_row_spec((BN, 2, hh)),
        out_shape=jax.ShapeDtypeStruct((n, 2, hh), jnp.float32),
    )(cur1, agg1_a, agg1_b, deg2d, s2, W1, b2(b1))

    agg2_a, agg2_b = sc_aggregate(row2d, col2d, cur2.reshape(2 * n, hh), deg)

    out = pl.pallas_call(
        _combine_body,
        grid=(nblk,),
        in_specs=[
            _row_spec((BN, 2, hh)), _row_spec((BN, hh)),
            _row_spec((BN, hh)), _row_spec((BN, 1)),
        ],
        out_specs=_row_spec((BN, h)),
        out_shape=jax.ShapeDtypeStruct((n, h), jnp.float32),
    )(cur2, agg2_a, agg2_b, deg2d)
    return out


# speedup vs baseline: 1.2827x; 1.2827x over previous
import functools

import jax
import jax.numpy as jnp
from jax import lax
from jax.experimental import pallas as pl
from jax.experimental.pallas import tpu as pltpu
from jax.experimental.pallas import tpu_sc as plsc

NC = 2
NS = 16
L = 16

BN = 400
CH = 80
SUP = 10
NBUF = 4


def _dense_layer1_body(texts, nf, Wt, bt, Wg01, bg01, Wg02, bg02,
                       Wg11, bg11, Wg12, bg12, W0, b0,
                       cur_out, s2_out):
    te = jnp.dot(texts[...], Wt[...], preferred_element_type=jnp.float32) + bt[...]
    t1 = jax.nn.relu(jnp.dot(te, Wg01[...], preferred_element_type=jnp.float32) + bg01[...])
    s1 = jax.nn.sigmoid(jnp.dot(t1, Wg02[...], preferred_element_type=jnp.float32) + bg02[...])
    c = (jnp.dot(nf[...], W0[...], preferred_element_type=jnp.float32) + b0[...]) * s1
    cur_out[...] = c.reshape(cur_out.shape)
    t2 = jax.nn.relu(jnp.dot(te, Wg11[...], preferred_element_type=jnp.float32) + bg11[...])
    s2_out[...] = jax.nn.sigmoid(jnp.dot(t2, Wg12[...], preferred_element_type=jnp.float32) + bg12[...])


def _dense_layer2_body(cur, agg_a, agg_b, deg, s2, W1, b1, cur2_out):
    bn = deg.shape[0]
    rdeg = 1.0 / jnp.maximum(deg[...], 1.0)
    agg = jnp.concatenate([agg_a[...], agg_b[...]], axis=1)
    c = cur[...].reshape(bn, -1)
    h = jax.nn.relu((c + agg * rdeg) * 0.5)
    c2 = (jnp.dot(h, W1[...], preferred_element_type=jnp.float32) + b1[...]) * s2[...]
    cur2_out[...] = c2.reshape(cur2_out.shape)


def _combine_body(cur, agg_a, agg_b, deg, out):
    bn = deg.shape[0]
    rdeg = 1.0 / jnp.maximum(deg[...], 1.0)
    agg = jnp.concatenate([agg_a[...], agg_b[...]], axis=1)
    out[...] = (cur[...].reshape(bn, -1) + agg * rdeg) * 0.5


def _row_spec(bshape):
    nd = len(bshape)
    return pl.BlockSpec(bshape, lambda i, _nd=nd: (i,) + (0,) * (_nd - 1))


def _full_spec(shape):
    nd = len(shape)
    return pl.BlockSpec(shape, lambda i, _nd=nd: (0,) * _nd)


_SC_PARAMS = pltpu.CompilerParams(use_tc_tiling_on_sc=False,
                                  needs_layout_passes=False)


def _mesh():
    return plsc.VectorSubcoreMesh(core_axis_name="c", subcore_axis_name="s",
                                  num_cores=NC, num_subcores=NS)


def _make_sc_degree(n, e, hh):
    ept = e // NS
    assert e % (NS * L) == 0
    degp = ((n + NS * hh - 1) // (NS * hh)) * NS * hh
    dgr = degp // hh
    drt = dgr // NS
    half = NS // 2

    @functools.partial(
        pl.kernel,
        out_type=jax.ShapeDtypeStruct((dgr, hh), jnp.float32),
        mesh=_mesh(),
        scratch_types=[
            pltpu.VMEM((ept,), jnp.int32),
            pltpu.VMEM((dgr, hh), jnp.float32),
            pltpu.VMEM((half * drt, hh), jnp.float32),
            pltpu.VMEM((drt, hh), jnp.float32),
            pltpu.HBM((NS * dgr, hh), jnp.float32),
        ],
        compiler_params=_SC_PARAMS,
    )
    def sc_degree(col_hbm, deg_hbm, colbuf, deg_v, red0, red1, deg_sh):
        cid = lax.axis_index("c")
        sid = lax.axis_index("s")
        zeros16 = jnp.zeros((L,), jnp.float32)
        ones16 = jnp.ones((L,), jnp.float32)

        @pl.when(cid == 0)
        def _():
            def _zdeg(i, _):
                for j in range(hh // L):
                    deg_v[i, pl.ds(j * L, L)] = zeros16
                return 0
            lax.fori_loop(0, dgr, _zdeg, 0)
            pltpu.sync_copy(col_hbm.at[pl.ds(sid * ept, ept)], colbuf)

            def _hist(i, _):
                for u in range(5):
                    c = colbuf[pl.ds((i * 5 + u) * L, L)]
                    plsc.addupdate_scatter(
                        deg_v,
                        [lax.shift_right_logical(c, 7),
                         lax.bitwise_and(c, 127)],
                        ones16)
                return 0
            lax.fori_loop(0, ept // (5 * L), _hist, 0)
            pltpu.sync_copy(deg_v, deg_sh.at[pl.ds(sid * dgr, dgr)])
            plsc.subcore_barrier()

            for g in range(2):
                for t8 in range(half):
                    tt = g * half + t8
                    pltpu.sync_copy(
                        deg_sh.at[pl.ds(tt * dgr + sid * drt, drt)],
                        red0.at[pl.ds(t8 * drt, drt)])
                for p in range(drt):
                    for q in range(hh // L):
                        acc = red0[p, pl.ds(q * L, L)]
                        for t8 in range(1, half):
                            acc = acc + red0[t8 * drt + p, pl.ds(q * L, L)]
                        if g == 0:
                            red1[p, pl.ds(q * L, L)] = acc
                        else:
                            red1[p, pl.ds(q * L, L)] = (
                                red1[p, pl.ds(q * L, L)] + acc)
                pltpu.sync_copy(red1, deg_hbm.at[pl.ds(sid * drt, drt)])

    return sc_degree


def _make_sc_aggregate(n, epad, hh):
    cpt = epad // (NS * CH)
    nsup = cpt // SUP
    rem = cpt % SUP
    assert epad % (NS * CH) == 0 and n % NS == 0 and hh == 128
    rpt = n // NS

    @functools.partial(
        pl.kernel,
        out_type=(
            jax.ShapeDtypeStruct((n, hh), jnp.float32),
            jax.ShapeDtypeStruct((n, hh), jnp.float32),
        ),
        mesh=_mesh(),
        scratch_types=(
            [pltpu.VMEM((SUP, CH), jnp.int32),
             pltpu.VMEM((SUP, CH), jnp.int32)]
            + [pltpu.VMEM((CH, hh), jnp.float32) for _ in range(NBUF)]
            + [pltpu.VMEM_SHARED((n + 8, hh), jnp.float32)]
            + [pltpu.SemaphoreType.DMA for _ in range(2 * NBUF)]
        ),
        compiler_params=_SC_PARAMS,
    )
    def sc_aggregate(row_hbm, col_hbm, cur_hbm, dep_hbm, agg_a_hbm,
                     agg_b_hbm, *rest):
        del dep_hbm
        rowbuf, colbuf = rest[0], rest[1]
        rb = rest[2:2 + NBUF]
        agg_sh = rest[2 + NBUF]
        gsem = rest[3 + NBUF:3 + 2 * NBUF]
        ssem = rest[3 + 2 * NBUF:3 + 3 * NBUF]
        cid = lax.axis_index("c")
        sid = lax.axis_index("s")
        zeros16 = jnp.zeros((L,), jnp.float32)

        def _zrow(i, _):
            for j in range(hh // L):
                rb[0][i, pl.ds(j * L, L)] = zeros16
            return 0
        lax.fori_loop(0, CH, _zrow, 0)
        done = 0
        while done < rpt:
            step = min(CH, rpt - done)
            pltpu.sync_copy(rb[0].at[pl.ds(0, step)],
                            agg_sh.at[pl.ds(sid * rpt + done, step)])
            done += step
        plsc.subcore_barrier()

        def _run_super(base, count):
            pltpu.sync_copy(row_hbm.at[pl.ds(base, count)],
                            rowbuf.at[pl.ds(0, count)])
            pltpu.sync_copy(col_hbm.at[pl.ds(base, count)],
                            colbuf.at[pl.ds(0, count)])
            for j in range(count):
                for l in range(CH // L):
                    r = rowbuf[j, pl.ds(l * L, L)]
                    rowbuf[j, pl.ds(l * L, L)] = r + r + cid
            gd = [None] * NBUF
            sd = [None] * NBUF
            for j in range(min(NBUF - 1, count)):
                gd[j] = pltpu.async_copy(cur_hbm.at[rowbuf.at[j]], rb[j],
                                         gsem[j])
            for j in range(count):
                b = j % NBUF
                gd[b].wait()
                if j + NBUF - 1 < count:
                    nb = (j + NBUF - 1) % NBUF
                    if sd[nb] is not None:
                        sd[nb].wait()
                    gd[nb] = pltpu.async_copy(
                        cur_hbm.at[rowbuf.at[j + NBUF - 1]], rb[nb], gsem[nb])
                sd[b] = pltpu.async_copy(rb[b], agg_sh.at[colbuf.at[j]],
                                         ssem[b], add=True)
            for b in range(NBUF):
                if sd[b] is not None:
                    sd[b].wait()

        def _super(s, _):
            _run_super(sid * cpt + s * SUP, SUP)
            return 0
        lax.fori_loop(0, nsup, _super, 0)
        if rem:
            _run_super(sid * cpt + nsup * SUP, rem)
        plsc.subcore_barrier()

        @pl.when(cid == 0)
        def _():
            pltpu.sync_copy(agg_sh.at[pl.ds(sid * rpt, rpt)],
                            agg_a_hbm.at[pl.ds(sid * rpt, rpt)])

        @pl.when(cid == 1)
        def _():
            pltpu.sync_copy(agg_sh.at[pl.ds(sid * rpt, rpt)],
                            agg_b_hbm.at[pl.ds(sid * rpt, rpt)])

    return sc_aggregate


def kernel(edge_index, node_features, node_texts, Wt, bt, Wg01, bg01, Wg02,
           bg02, Wg11, bg11, Wg12, bg12, W0, b0, W1, b1):
    n, d = node_features.shape
    t = node_texts.shape[1]
    h = W0.shape[1]
    e = edge_index.shape[1]
    hh = h // NC
    assert n % BN == 0
    nblk = n // BN

    row = edge_index[0]
    col = edge_index[1]
    epad = ((e + NS * CH - 1) // (NS * CH)) * NS * CH
    if epad != e:
        rowp = jnp.concatenate([row, jnp.zeros((epad - e,), jnp.int32)])
        colp = jnp.concatenate([col, jnp.full((epad - e,), n, jnp.int32)])
    else:
        rowp, colp = row, col
    row2d = rowp.reshape(epad // CH, CH)
    col2d = colp.reshape(epad // CH, CH)
    b2 = lambda b: b.reshape(1, -1)

    sc_degree = _make_sc_degree(n, e, hh)
    deg = sc_degree(col)
    deg2d = deg.reshape(-1)[:n].reshape(n, 1)

    cur1, s2 = pl.pallas_call(
        _dense_layer1_body,
        grid=(nblk,),
        in_specs=[
            _row_spec((BN, t)), _row_spec((BN, d)),
            _full_spec((t, h)), _full_spec((1, h)),
            _full_spec((h, h)), _full_spec((1, h)),
            _full_spec((h, h)), _full_spec((1, h)),
            _full_spec((h, h)), _full_spec((1, h)),
            _full_spec((h, h)), _full_spec((1, h)),
            _full_spec((d, h)), _full_spec((1, h)),
        ],
        out_specs=[_row_spec((BN, 2, hh)), _row_spec((BN, h))],
        out_shape=[
            jax.ShapeDtypeStruct((n, 2, hh), jnp.float32),
            jax.ShapeDtypeStruct((n, h), jnp.float32),
        ],
    )(node_texts, node_features, Wt, b2(bt), Wg01, b2(bg01), Wg02, b2(bg02),
      Wg11, b2(bg11), Wg12, b2(bg12), W0, b2(b0))

    sc_aggregate = _make_sc_aggregate(n, epad, hh)

    agg1_a, agg1_b = sc_aggregate(row2d, col2d, cur1.reshape(2 * n, hh), deg)

    cur2 = pl.pallas_call(
        _dense_layer2_body,
        grid=(nblk,),
        in_specs=[
            _row_spec((BN, 2, hh)), _row_spec((BN, hh)),
            _row_spec((BN, hh)), _row_spec((BN, 1)), _row_spec((BN, h)),
            _full_spec((h, h)), _full_spec((1, h)),
        ],
        out_specs=_row_spec((BN, 2, hh)),
        out_shape=jax.ShapeDtypeStruct((n, 2, hh), jnp.float32),
    )(cur1, agg1_a, agg1_b, deg2d, s2, W1, b2(b1))

    agg2_a, agg2_b = sc_aggregate(row2d, col2d, cur2.reshape(2 * n, hh), deg)

    out = pl.pallas_call(
        _combine_body,
        grid=(nblk,),
        in_specs=[
            _row_spec((BN, 2, hh)), _row_spec((BN, hh)),
            _row_spec((BN, hh)), _row_spec((BN, 1)),
        ],
        out_specs=_row_spec((BN, h)),
        out_shape=jax.ShapeDtypeStruct((n, h), jnp.float32),
    )(cur2, agg2_a, agg2_b, deg2d)
    return out


# speedup vs baseline: 1.4033x; 1.0940x over previous
import functools

import jax
import jax.numpy as jnp
from jax import lax
from jax.experimental import pallas as pl
from jax.experimental.pallas import tpu as pltpu
from jax.experimental.pallas import tpu_sc as plsc

NC = 2
NS = 16
L = 16

BN = 400
CH = 80
SUP = 10
NBUF = 4


def _dense_layer1_body(texts, nf, Wt, bt, Wg01, bg01, Wg02, bg02,
                       Wg11, bg11, Wg12, bg12, W0, b0,
                       cur_out, s2_out):
    te = jnp.dot(texts[...], Wt[...], preferred_element_type=jnp.float32) + bt[...]
    t1 = jax.nn.relu(jnp.dot(te, Wg01[...], preferred_element_type=jnp.float32) + bg01[...])
    s1 = jax.nn.sigmoid(jnp.dot(t1, Wg02[...], preferred_element_type=jnp.float32) + bg02[...])
    c = (jnp.dot(nf[...], W0[...], preferred_element_type=jnp.float32) + b0[...]) * s1
    cur_out[...] = c.reshape(cur_out.shape)
    t2 = jax.nn.relu(jnp.dot(te, Wg11[...], preferred_element_type=jnp.float32) + bg11[...])
    s2_out[...] = jax.nn.sigmoid(jnp.dot(t2, Wg12[...], preferred_element_type=jnp.float32) + bg12[...])


def _dense_layer2_body(cur, agg_a, agg_b, deg, s2, W1, b1, cur2_out):
    bn = deg.shape[0]
    rdeg = 1.0 / jnp.maximum(deg[...], 1.0)
    agg = jnp.concatenate([agg_a[...], agg_b[...]], axis=1)
    c = cur[...].reshape(bn, -1)
    h = jax.nn.relu((c + agg * rdeg) * 0.5)
    c2 = (jnp.dot(h, W1[...], preferred_element_type=jnp.float32) + b1[...]) * s2[...]
    cur2_out[...] = c2.reshape(cur2_out.shape)


def _combine_body(cur, agg_a, agg_b, deg, out):
    bn = deg.shape[0]
    rdeg = 1.0 / jnp.maximum(deg[...], 1.0)
    agg = jnp.concatenate([agg_a[...], agg_b[...]], axis=1)
    out[...] = (cur[...].reshape(bn, -1) + agg * rdeg) * 0.5


def _row_spec(bshape):
    nd = len(bshape)
    return pl.BlockSpec(bshape, lambda i, _nd=nd: (i,) + (0,) * (_nd - 1))


def _full_spec(shape):
    nd = len(shape)
    return pl.BlockSpec(shape, lambda i, _nd=nd: (0,) * _nd)


_SC_PARAMS = pltpu.CompilerParams(use_tc_tiling_on_sc=False,
                                  needs_layout_passes=False)


def _mesh():
    return plsc.VectorSubcoreMesh(core_axis_name="c", subcore_axis_name="s",
                                  num_cores=NC, num_subcores=NS)


def _make_sc_degree(n, e, hh):
    ept = e // NS
    assert e % (NS * L) == 0
    degp = ((n + NS * hh - 1) // (NS * hh)) * NS * hh
    dgr = degp // hh
    drt = dgr // NS
    half = NS // 2

    @functools.partial(
        pl.kernel,
        out_type=jax.ShapeDtypeStruct((dgr, hh), jnp.float32),
        mesh=_mesh(),
        scratch_types=[
            pltpu.VMEM((ept,), jnp.int32),
            pltpu.VMEM((dgr, hh), jnp.float32),
            pltpu.VMEM((half * drt, hh), jnp.float32),
            pltpu.VMEM((drt, hh), jnp.float32),
            pltpu.HBM((NS * dgr, hh), jnp.float32),
        ],
        compiler_params=_SC_PARAMS,
    )
    def sc_degree(col_hbm, deg_hbm, colbuf, deg_v, red0, red1, deg_sh):
        cid = lax.axis_index("c")
        sid = lax.axis_index("s")
        zeros16 = jnp.zeros((L,), jnp.float32)
        ones16 = jnp.ones((L,), jnp.float32)

        @pl.when(cid == 0)
        def _():
            def _zdeg(i, _):
                for j in range(hh // L):
                    deg_v[i, pl.ds(j * L, L)] = zeros16
                return 0
            lax.fori_loop(0, dgr, _zdeg, 0)
            pltpu.sync_copy(col_hbm.at[pl.ds(sid * ept, ept)], colbuf)

            def _hist(i, _):
                for u in range(5):
                    c = colbuf[pl.ds((i * 5 + u) * L, L)]
                    plsc.addupdate_scatter(
                        deg_v,
                        [lax.shift_right_logical(c, 7),
                         lax.bitwise_and(c, 127)],
                        ones16)
                return 0
            lax.fori_loop(0, ept // (5 * L), _hist, 0)
            pltpu.sync_copy(deg_v, deg_sh.at[pl.ds(sid * dgr, dgr)])
            plsc.subcore_barrier()

            for g in range(2):
                for t8 in range(half):
                    tt = g * half + t8
                    pltpu.sync_copy(
                        deg_sh.at[pl.ds(tt * dgr + sid * drt, drt)],
                        red0.at[pl.ds(t8 * drt, drt)])
                for p in range(drt):
                    for q in range(hh // L):
                        acc = red0[p, pl.ds(q * L, L)]
                        for t8 in range(1, half):
                            acc = acc + red0[t8 * drt + p, pl.ds(q * L, L)]
                        if g == 0:
                            red1[p, pl.ds(q * L, L)] = acc
                        else:
                            red1[p, pl.ds(q * L, L)] = (
                                red1[p, pl.ds(q * L, L)] + acc)
                pltpu.sync_copy(red1, deg_hbm.at[pl.ds(sid * drt, drt)])

    return sc_degree


def _make_sc_aggregate(n, epad, hh):
    cpt = epad // (NS * CH)
    nsup = cpt // SUP
    assert epad % (NS * CH) == 0 and cpt % SUP == 0 and n % NS == 0
    assert hh == 128 and nsup >= 2
    npair = nsup // 2
    tail = nsup - 2 * npair
    rpt = n // NS

    @functools.partial(
        pl.kernel,
        out_type=(
            jax.ShapeDtypeStruct((n, hh), jnp.float32),
            jax.ShapeDtypeStruct((n, hh), jnp.float32),
        ),
        mesh=_mesh(),
        scratch_types=(
            [pltpu.VMEM((SUP, CH), jnp.int32) for _ in range(4)]
            + [pltpu.VMEM((CH, hh), jnp.float32) for _ in range(NBUF)]
            + [pltpu.VMEM_SHARED((n + 8, hh), jnp.float32)]
            + [pltpu.SemaphoreType.DMA for _ in range(2 * NBUF + 4)]
        ),
        compiler_params=_SC_PARAMS,
    )
    def sc_aggregate(row_hbm, col_hbm, cur_hbm, dep_hbm, agg_a_hbm,
                     agg_b_hbm, *rest):
        del dep_hbm
        rowb = rest[0:2]
        colb = rest[2:4]
        rb = rest[4:4 + NBUF]
        agg_sh = rest[4 + NBUF]
        gsem = rest[5 + NBUF:5 + 2 * NBUF]
        ssem = rest[5 + 2 * NBUF:5 + 3 * NBUF]
        irsem = rest[5 + 3 * NBUF:7 + 3 * NBUF]
        icsem = rest[7 + 3 * NBUF:9 + 3 * NBUF]
        cid = lax.axis_index("c")
        sid = lax.axis_index("s")
        zeros16 = jnp.zeros((L,), jnp.float32)

        def _zrow(i, _):
            for j in range(hh // L):
                rb[0][i, pl.ds(j * L, L)] = zeros16
            return 0
        lax.fori_loop(0, CH, _zrow, 0)
        done = 0
        while done < rpt:
            step = min(CH, rpt - done)
            pltpu.sync_copy(rb[0].at[pl.ds(0, step)],
                            agg_sh.at[pl.ds(sid * rpt + done, step)])
            done += step
        plsc.subcore_barrier()

        def _idx_issue(s, par):
            base = sid * cpt + s * SUP
            pltpu.async_copy(row_hbm.at[pl.ds(base, SUP)], rowb[par],
                             irsem[par])
            pltpu.async_copy(col_hbm.at[pl.ds(base, SUP)], colb[par],
                             icsem[par])

        def _idx_wait(par):
            pltpu.make_async_copy(row_hbm.at[pl.ds(0, SUP)], rowb[par],
                                  irsem[par]).wait()
            pltpu.make_async_copy(col_hbm.at[pl.ds(0, SUP)], colb[par],
                                  icsem[par]).wait()

        def _run_super(par):
            rowbuf, colbuf = rowb[par], colb[par]
            for j in range(SUP):
                for l in range(CH // L):
                    r = rowbuf[j, pl.ds(l * L, L)]
                    rowbuf[j, pl.ds(l * L, L)] = r + r + cid
            gd = [None] * NBUF
            sd = [None] * NBUF
            for j in range(min(NBUF - 1, SUP)):
                gd[j] = pltpu.async_copy(cur_hbm.at[rowbuf.at[j]], rb[j],
                                         gsem[j])
            for j in range(SUP):
                b = j % NBUF
                gd[b].wait()
                if j + NBUF - 1 < SUP:
                    nb = (j + NBUF - 1) % NBUF
                    if sd[nb] is not None:
                        sd[nb].wait()
                    gd[nb] = pltpu.async_copy(
                        cur_hbm.at[rowbuf.at[j + NBUF - 1]], rb[nb], gsem[nb])
                sd[b] = pltpu.async_copy(rb[b], agg_sh.at[colbuf.at[j]],
                                         ssem[b], add=True)
            for b in range(NBUF):
                if sd[b] is not None:
                    sd[b].wait()

        _idx_issue(0, 0)
        _idx_issue(1, 1)

        def _pair(tp, _):
            for par in range(2):
                s = 2 * tp + par
                _idx_wait(par)
                _run_super(par)
                nxt = s + 2

                @pl.when(nxt < nsup)
                def _():
                    _idx_issue(nxt, par)
            return 0
        lax.fori_loop(0, npair, _pair, 0)
        if tail:
            par = (nsup - 1) % 2
            _idx_wait(par)
            _run_super(par)
        plsc.subcore_barrier()

        @pl.when(cid == 0)
        def _():
            pltpu.sync_copy(agg_sh.at[pl.ds(sid * rpt, rpt)],
                            agg_a_hbm.at[pl.ds(sid * rpt, rpt)])

        @pl.when(cid == 1)
        def _():
            pltpu.sync_copy(agg_sh.at[pl.ds(sid * rpt, rpt)],
                            agg_b_hbm.at[pl.ds(sid * rpt, rpt)])

    return sc_aggregate


def kernel(edge_index, node_features, node_texts, Wt, bt, Wg01, bg01, Wg02,
           bg02, Wg11, bg11, Wg12, bg12, W0, b0, W1, b1):
    n, d = node_features.shape
    t = node_texts.shape[1]
    h = W0.shape[1]
    e = edge_index.shape[1]
    hh = h // NC
    assert n % BN == 0
    nblk = n // BN

    row = edge_index[0]
    col = edge_index[1]
    epad = ((e + NS * CH - 1) // (NS * CH)) * NS * CH
    if epad != e:
        rowp = jnp.concatenate([row, jnp.zeros((epad - e,), jnp.int32)])
        colp = jnp.concatenate([col, jnp.full((epad - e,), n, jnp.int32)])
    else:
        rowp, colp = row, col
    row2d = rowp.reshape(epad // CH, CH)
    col2d = colp.reshape(epad // CH, CH)
    b2 = lambda b: b.reshape(1, -1)

    sc_degree = _make_sc_degree(n, e, hh)
    deg = sc_degree(col)
    deg2d = deg.reshape(-1)[:n].reshape(n, 1)

    cur1, s2 = pl.pallas_call(
        _dense_layer1_body,
        grid=(nblk,),
        in_specs=[
            _row_spec((BN, t)), _row_spec((BN, d)),
            _full_spec((t, h)), _full_spec((1, h)),
            _full_spec((h, h)), _full_spec((1, h)),
            _full_spec((h, h)), _full_spec((1, h)),
            _full_spec((h, h)), _full_spec((1, h)),
            _full_spec((h, h)), _full_spec((1, h)),
            _full_spec((d, h)), _full_spec((1, h)),
        ],
        out_specs=[_row_spec((BN, 2, hh)), _row_spec((BN, h))],
        out_shape=[
            jax.ShapeDtypeStruct((n, 2, hh), jnp.float32),
            jax.ShapeDtypeStruct((n, h), jnp.float32),
        ],
    )(node_texts, node_features, Wt, b2(bt), Wg01, b2(bg01), Wg02, b2(bg02),
      Wg11, b2(bg11), Wg12, b2(bg12), W0, b2(b0))

    sc_aggregate = _make_sc_aggregate(n, epad, hh)

    agg1_a, agg1_b = sc_aggregate(row2d, col2d, cur1.reshape(2 * n, hh), deg)

    cur2 = pl.pallas_call(
        _dense_layer2_body,
        grid=(nblk,),
        in_specs=[
            _row_spec((BN, 2, hh)), _row_spec((BN, hh)),
            _row_spec((BN, hh)), _row_spec((BN, 1)), _row_spec((BN, h)),
            _full_spec((h, h)), _full_spec((1, h)),
        ],
        out_specs=_row_spec((BN, 2, hh)),
        out_shape=jax.ShapeDtypeStruct((n, 2, hh), jnp.float32),
    )(cur1, agg1_a, agg1_b, deg2d, s2, W1, b2(b1))

    agg2_a, agg2_b = sc_aggregate(row2d, col2d, cur2.reshape(2 * n, hh), deg)

    out = pl.pallas_call(
        _combine_body,
        grid=(nblk,),
        in_specs=[
            _row_spec((BN, 2, hh)), _row_spec((BN, hh)),
            _row_spec((BN, hh)), _row_spec((BN, 1)),
        ],
        out_specs=_row_spec((BN, h)),
        out_shape=jax.ShapeDtypeStruct((n, h), jnp.float32),
    )(cur2, agg2_a, agg2_b, deg2d)
    return out


# speedup vs baseline: 1.4668x; 1.0453x over previous
import functools

import jax
import jax.numpy as jnp
from jax import lax
from jax.experimental import pallas as pl
from jax.experimental.pallas import tpu as pltpu
from jax.experimental.pallas import tpu_sc as plsc

NC = 2
NS = 16
L = 16

BN = 1000
CH = 80
SUP = 10
NBUF = 4


def _dense_layer1_body(texts, nf, Wt, bt, Wg01, bg01, Wg02, bg02,
                       Wg11, bg11, Wg12, bg12, W0, b0,
                       cur_out, s2_out):
    te = jnp.dot(texts[...], Wt[...], preferred_element_type=jnp.float32) + bt[...]
    t1 = jax.nn.relu(jnp.dot(te, Wg01[...], preferred_element_type=jnp.float32) + bg01[...])
    s1 = jax.nn.sigmoid(jnp.dot(t1, Wg02[...], preferred_element_type=jnp.float32) + bg02[...])
    c = (jnp.dot(nf[...], W0[...], preferred_element_type=jnp.float32) + b0[...]) * s1
    cur_out[...] = c.reshape(cur_out.shape)
    t2 = jax.nn.relu(jnp.dot(te, Wg11[...], preferred_element_type=jnp.float32) + bg11[...])
    s2_out[...] = jax.nn.sigmoid(jnp.dot(t2, Wg12[...], preferred_element_type=jnp.float32) + bg12[...])


def _dense_layer2_body(cur, agg_a, agg_b, deg, s2, W1, b1, cur2_out):
    bn = deg.shape[0]
    rdeg = 1.0 / jnp.maximum(deg[...], 1.0)
    agg = jnp.concatenate([agg_a[...], agg_b[...]], axis=1)
    c = cur[...].reshape(bn, -1)
    h = jax.nn.relu((c + agg * rdeg) * 0.5)
    c2 = (jnp.dot(h, W1[...], preferred_element_type=jnp.float32) + b1[...]) * s2[...]
    cur2_out[...] = c2.reshape(cur2_out.shape)


def _combine_body(cur, agg_a, agg_b, deg, out):
    bn = deg.shape[0]
    rdeg = 1.0 / jnp.maximum(deg[...], 1.0)
    agg = jnp.concatenate([agg_a[...], agg_b[...]], axis=1)
    out[...] = (cur[...].reshape(bn, -1) + agg * rdeg) * 0.5


def _row_spec(bshape):
    nd = len(bshape)
    return pl.BlockSpec(bshape, lambda i, _nd=nd: (i,) + (0,) * (_nd - 1))


def _full_spec(shape):
    nd = len(shape)
    return pl.BlockSpec(shape, lambda i, _nd=nd: (0,) * _nd)


_SC_PARAMS = pltpu.CompilerParams(use_tc_tiling_on_sc=False,
                                  needs_layout_passes=False)


def _mesh():
    return plsc.VectorSubcoreMesh(core_axis_name="c", subcore_axis_name="s",
                                  num_cores=NC, num_subcores=NS)


def _make_sc_degree(n, e, hh):
    ept = e // NS
    assert e % (NS * L) == 0
    degp = ((n + NS * hh - 1) // (NS * hh)) * NS * hh
    dgr = degp // hh
    drt = dgr // NS
    half = NS // 2

    @functools.partial(
        pl.kernel,
        out_type=jax.ShapeDtypeStruct((dgr, hh), jnp.float32),
        mesh=_mesh(),
        scratch_types=[
            pltpu.VMEM((ept,), jnp.int32),
            pltpu.VMEM((dgr, hh), jnp.float32),
            pltpu.VMEM((half * drt, hh), jnp.float32),
            pltpu.VMEM((drt, hh), jnp.float32),
            pltpu.HBM((NS * dgr, hh), jnp.float32),
        ],
        compiler_params=_SC_PARAMS,
    )
    def sc_degree(col_hbm, deg_hbm, colbuf, deg_v, red0, red1, deg_sh):
        cid = lax.axis_index("c")
        sid = lax.axis_index("s")
        zeros16 = jnp.zeros((L,), jnp.float32)
        ones16 = jnp.ones((L,), jnp.float32)

        @pl.when(cid == 0)
        def _():
            def _zdeg(i, _):
                for j in range(hh // L):
                    deg_v[i, pl.ds(j * L, L)] = zeros16
                return 0
            lax.fori_loop(0, dgr, _zdeg, 0)
            pltpu.sync_copy(col_hbm.at[pl.ds(sid * ept, ept)], colbuf)

            def _hist(i, _):
                for u in range(5):
                    c = colbuf[pl.ds((i * 5 + u) * L, L)]
                    plsc.addupdate_scatter(
                        deg_v,
                        [lax.shift_right_logical(c, 7),
                         lax.bitwise_and(c, 127)],
                        ones16)
                return 0
            lax.fori_loop(0, ept // (5 * L), _hist, 0)
            pltpu.sync_copy(deg_v, deg_sh.at[pl.ds(sid * dgr, dgr)])
            plsc.subcore_barrier()

            for g in range(2):
                for t8 in range(half):
                    tt = g * half + t8
                    pltpu.sync_copy(
                        deg_sh.at[pl.ds(tt * dgr + sid * drt, drt)],
                        red0.at[pl.ds(t8 * drt, drt)])
                for p in range(drt):
                    for q in range(hh // L):
                        acc = red0[p, pl.ds(q * L, L)]
                        for t8 in range(1, half):
                            acc = acc + red0[t8 * drt + p, pl.ds(q * L, L)]
                        if g == 0:
                            red1[p, pl.ds(q * L, L)] = acc
                        else:
                            red1[p, pl.ds(q * L, L)] = (
                                red1[p, pl.ds(q * L, L)] + acc)
                pltpu.sync_copy(red1, deg_hbm.at[pl.ds(sid * drt, drt)])

    return sc_degree


def _make_sc_aggregate(n, epad, hh):
    cpt = epad // (NS * CH)
    nsup = cpt // SUP
    assert epad % (NS * CH) == 0 and cpt % SUP == 0 and n % NS == 0
    assert hh == 128 and nsup >= 2
    npair = nsup // 2
    tail = nsup - 2 * npair
    rpt = n // NS

    @functools.partial(
        pl.kernel,
        out_type=(
            jax.ShapeDtypeStruct((n, hh), jnp.float32),
            jax.ShapeDtypeStruct((n, hh), jnp.float32),
        ),
        mesh=_mesh(),
        scratch_types=(
            [pltpu.VMEM((SUP, CH), jnp.int32) for _ in range(4)]
            + [pltpu.VMEM((CH, hh), jnp.float32) for _ in range(NBUF)]
            + [pltpu.VMEM_SHARED((n + 8, hh), jnp.float32)]
            + [pltpu.SemaphoreType.DMA for _ in range(2 * NBUF + 4)]
        ),
        compiler_params=_SC_PARAMS,
    )
    def sc_aggregate(row_hbm, col_hbm, cur_hbm, dep_hbm, agg_a_hbm,
                     agg_b_hbm, *rest):
        del dep_hbm
        rowb = rest[0:2]
        colb = rest[2:4]
        rb = rest[4:4 + NBUF]
        agg_sh = rest[4 + NBUF]
        gsem = rest[5 + NBUF:5 + 2 * NBUF]
        ssem = rest[5 + 2 * NBUF:5 + 3 * NBUF]
        irsem = rest[5 + 3 * NBUF:7 + 3 * NBUF]
        icsem = rest[7 + 3 * NBUF:9 + 3 * NBUF]
        cid = lax.axis_index("c")
        sid = lax.axis_index("s")
        zeros16 = jnp.zeros((L,), jnp.float32)

        def _zrow(i, _):
            for j in range(hh // L):
                rb[0][i, pl.ds(j * L, L)] = zeros16
            return 0
        lax.fori_loop(0, CH, _zrow, 0)
        done = 0
        while done < rpt:
            step = min(CH, rpt - done)
            pltpu.sync_copy(rb[0].at[pl.ds(0, step)],
                            agg_sh.at[pl.ds(sid * rpt + done, step)])
            done += step
        plsc.subcore_barrier()

        def _idx_issue(s, par):
            base = sid * cpt + s * SUP
            pltpu.async_copy(row_hbm.at[pl.ds(base, SUP)], rowb[par],
                             irsem[par])
            pltpu.async_copy(col_hbm.at[pl.ds(base, SUP)], colb[par],
                             icsem[par])

        def _idx_wait(par):
            pltpu.make_async_copy(row_hbm.at[pl.ds(0, SUP)], rowb[par],
                                  irsem[par]).wait()
            pltpu.make_async_copy(col_hbm.at[pl.ds(0, SUP)], colb[par],
                                  icsem[par]).wait()

        def _run_super(par):
            rowbuf, colbuf = rowb[par], colb[par]
            for j in range(SUP):
                for l in range(CH // L):
                    r = rowbuf[j, pl.ds(l * L, L)]
                    rowbuf[j, pl.ds(l * L, L)] = r + r + cid
            gd = [None] * NBUF
            sd = [None] * NBUF
            for j in range(min(NBUF - 1, SUP)):
                gd[j] = pltpu.async_copy(cur_hbm.at[rowbuf.at[j]], rb[j],
                                         gsem[j])
            for j in range(SUP):
                b = j % NBUF
                gd[b].wait()
                if j + NBUF - 1 < SUP:
                    nb = (j + NBUF - 1) % NBUF
                    if sd[nb] is not None:
                        sd[nb].wait()
                    gd[nb] = pltpu.async_copy(
                        cur_hbm.at[rowbuf.at[j + NBUF - 1]], rb[nb], gsem[nb])
                sd[b] = pltpu.async_copy(rb[b], agg_sh.at[colbuf.at[j]],
                                         ssem[b], add=True)
            for b in range(NBUF):
                if sd[b] is not None:
                    sd[b].wait()

        _idx_issue(0, 0)
        _idx_issue(1, 1)

        def _pair(tp, _):
            for par in range(2):
                s = 2 * tp + par
                _idx_wait(par)
                _run_super(par)
                nxt = s + 2

                @pl.when(nxt < nsup)
                def _():
                    _idx_issue(nxt, par)
            return 0
        lax.fori_loop(0, npair, _pair, 0)
        if tail:
            par = (nsup - 1) % 2
            _idx_wait(par)
            _run_super(par)
        plsc.subcore_barrier()

        @pl.when(cid == 0)
        def _():
            pltpu.sync_copy(agg_sh.at[pl.ds(sid * rpt, rpt)],
                            agg_a_hbm.at[pl.ds(sid * rpt, rpt)])

        @pl.when(cid == 1)
        def _():
            pltpu.sync_copy(agg_sh.at[pl.ds(sid * rpt, rpt)],
                            agg_b_hbm.at[pl.ds(sid * rpt, rpt)])

    return sc_aggregate


def kernel(edge_index, node_features, node_texts, Wt, bt, Wg01, bg01, Wg02,
           bg02, Wg11, bg11, Wg12, bg12, W0, b0, W1, b1):
    n, d = node_features.shape
    t = node_texts.shape[1]
    h = W0.shape[1]
    e = edge_index.shape[1]
    hh = h // NC
    assert n % BN == 0
    nblk = n // BN

    row = edge_index[0]
    col = edge_index[1]
    epad = ((e + NS * CH - 1) // (NS * CH)) * NS * CH
    if epad != e:
        rowp = jnp.concatenate([row, jnp.zeros((epad - e,), jnp.int32)])
        colp = jnp.concatenate([col, jnp.full((epad - e,), n, jnp.int32)])
    else:
        rowp, colp = row, col
    row2d = rowp.reshape(epad // CH, CH)
    col2d = colp.reshape(epad // CH, CH)
    b2 = lambda b: b.reshape(1, -1)

    sc_degree = _make_sc_degree(n, e, hh)
    deg = sc_degree(col)
    deg2d = deg.reshape(-1)[:n].reshape(n, 1)

    cur1, s2 = pl.pallas_call(
        _dense_layer1_body,
        grid=(nblk,),
        in_specs=[
            _row_spec((BN, t)), _row_spec((BN, d)),
            _full_spec((t, h)), _full_spec((1, h)),
            _full_spec((h, h)), _full_spec((1, h)),
            _full_spec((h, h)), _full_spec((1, h)),
            _full_spec((h, h)), _full_spec((1, h)),
            _full_spec((h, h)), _full_spec((1, h)),
            _full_spec((d, h)), _full_spec((1, h)),
        ],
        out_specs=[_row_spec((BN, 2, hh)), _row_spec((BN, h))],
        out_shape=[
            jax.ShapeDtypeStruct((n, 2, hh), jnp.float32),
            jax.ShapeDtypeStruct((n, h), jnp.float32),
        ],
    )(node_texts, node_features, Wt, b2(bt), Wg01, b2(bg01), Wg02, b2(bg02),
      Wg11, b2(bg11), Wg12, b2(bg12), W0, b2(b0))

    sc_aggregate = _make_sc_aggregate(n, epad, hh)

    agg1_a, agg1_b = sc_aggregate(row2d, col2d, cur1.reshape(2 * n, hh), deg)

    cur2 = pl.pallas_call(
        _dense_layer2_body,
        grid=(nblk,),
        in_specs=[
            _row_spec((BN, 2, hh)), _row_spec((BN, hh)),
            _row_spec((BN, hh)), _row_spec((BN, 1)), _row_spec((BN, h)),
            _full_spec((h, h)), _full_spec((1, h)),
        ],
        out_specs=_row_spec((BN, 2, hh)),
        out_shape=jax.ShapeDtypeStruct((n, 2, hh), jnp.float32),
    )(cur1, agg1_a, agg1_b, deg2d, s2, W1, b2(b1))

    agg2_a, agg2_b = sc_aggregate(row2d, col2d, cur2.reshape(2 * n, hh), deg)

    out = pl.pallas_call(
        _combine_body,
        grid=(nblk,),
        in_specs=[
            _row_spec((BN, 2, hh)), _row_spec((BN, hh)),
            _row_spec((BN, hh)), _row_spec((BN, 1)),
        ],
        out_specs=_row_spec((BN, h)),
        out_shape=jax.ShapeDtypeStruct((n, h), jnp.float32),
    )(cur2, agg2_a, agg2_b, deg2d)
    return out
